# Initial kernel scaffold; baseline (speedup 1.0000x reference)
import jax
import jax.numpy as jnp
from jax.experimental import pallas as pl


def kernel(user_id, neg_item_ids, user_emb_table, item_emb_table):
    raise NotImplementedError("write your pallas kernel here")


# speedup vs baseline: 1.8935x; 1.8935x over previous
import jax
import jax.numpy as jnp
import numpy as np
from jax import lax
from jax.experimental import pallas as pl
from jax.experimental.pallas import tpu as pltpu
from jax.experimental.pallas import tpu_sc as plsc

B = 4096
N = 200
D = 128
DP = D // 2
NC = 2
NS = 16
L = 16
NW = NC * NS
UPW = B // NW
NG = 13
NEG_INF = float("-inf")
HI_MASK = np.int32(np.uint32(0xFFFF0000))


def _bf16_split(w):
    even = plsc.bitcast(w << jnp.int32(16), jnp.float32)
    odd = plsc.bitcast(w & HI_MASK, jnp.float32)
    return even, odd


def _round_bf16(x):
    p = plsc.bitcast(x, jnp.int32)
    p = p + jnp.int32(0x7FFF) + ((p >> jnp.int32(16)) & jnp.int32(1))
    return plsc.bitcast(p & HI_MASK, jnp.float32)


def _sc_body(user_id_hbm, neg_flat_hbm, user_tab_hbm, item_tab_hbm,
             scores_out_hbm, sel_out_hbm,
             uid_v, urows_v, ids_v, rows_v, scores_v, selid_v,
             sem, sem2):
    wid = lax.axis_index("s") * NC + lax.axis_index("c")
    base_u = wid * UPW

    pltpu.sync_copy(user_id_hbm.at[pl.ds(base_u, UPW)], uid_v)
    pltpu.async_copy(user_tab_hbm.at[uid_v], urows_v, sem).wait()

    def round_user(ui, _):
        for k in range(D // L):
            urows_v[ui, pl.ds(k * L, L)] = _round_bf16(urows_v[ui, pl.ds(k * L, L)])
        return _
    lax.fori_loop(0, UPW, round_user, None)

    lanes = lax.iota(jnp.int32, L)
    n_vecs = [lanes + jnp.int32(g * L) for g in range(NG)]
    lane_lt8 = lanes < 8

    def user_body(u, _):
        b = base_u + u
        pltpu.sync_copy(neg_flat_hbm.at[pl.ds(b * N, N)], ids_v.at[pl.ds(0, N)])
        c1 = pltpu.async_copy(item_tab_hbm.at[ids_v.at[pl.ds(0, 128)]],
                              rows_v.at[pl.ds(0, 128)], sem)
        c2 = pltpu.async_copy(item_tab_hbm.at[ids_v.at[pl.ds(128, N - 128)]],
                              rows_v.at[pl.ds(128, N - 128)], sem2)
        c1.wait()
        c2.wait()

        u_splat = jnp.full((L,), u, jnp.int32)

        def d_body(dp, accs):
            dp_splat = jnp.full((L,), dp, jnp.int32)
            ue = plsc.load_gather(urows_v, [u_splat, dp_splat * 2])
            uo = plsc.load_gather(urows_v, [u_splat, dp_splat * 2 + 1])
            out = []
            for g, acc in enumerate(accs):
                w = plsc.load_gather(rows_v, [n_vecs[g], dp_splat])
                e, o = _bf16_split(w)
                out.append(acc + e * ue + o * uo)
            return tuple(out)

        accs = lax.fori_loop(
            0, DP, d_body, tuple(jnp.zeros((L,), jnp.float32) for _ in range(NG)))

        cur_max = jnp.full((L,), NEG_INF)
        cur_n = jnp.full((L,), 2147483647, jnp.int32)
        cur_id = jnp.zeros((L,), jnp.int32)
        for g in range(NG):
            s_g = accs[g]
            if g == NG - 1:
                s_g = jnp.where(lane_lt8, s_g, NEG_INF)
            ids_g = ids_v[pl.ds(g * L, L)]
            upd = s_g > cur_max
            cur_max = jnp.where(upd, s_g, cur_max)
            cur_n = jnp.where(upd, n_vecs[g], cur_n)
            cur_id = jnp.where(upd, ids_g, cur_id)
            scores_v[pl.ds(g * L, L)] = s_g

        m = jnp.max(cur_max)
        big = jnp.int32(2147483647)
        n_sel = jnp.min(jnp.where(cur_max == m, cur_n, big))
        id_sel = jnp.min(jnp.where(cur_n == n_sel, cur_id, big))
        plsc.store_scatter(selid_v, [u_splat],
                           jnp.full((L,), id_sel, jnp.int32),
                           mask=lanes == 0)

        pltpu.sync_copy(scores_v.at[pl.ds(0, N)],
                        scores_out_hbm.at[pl.ds(b * N, N)])
        return _

    lax.fori_loop(0, UPW, user_body, None)
    pltpu.sync_copy(selid_v, sel_out_hbm.at[pl.ds(base_u, UPW)])


@jax.jit
def _run(user_id, neg_flat, user_tab, item_tab_packed):
    mesh = plsc.VectorSubcoreMesh(core_axis_name="c", subcore_axis_name="s",
                                  num_cores=NC, num_subcores=NS)
    f = pl.kernel(
        _sc_body,
        out_type=(
            jax.ShapeDtypeStruct((B * N,), jnp.float32),
            jax.ShapeDtypeStruct((B,), jnp.int32),
        ),
        mesh=mesh,
        compiler_params=pltpu.CompilerParams(needs_layout_passes=False,
                                             use_tc_tiling_on_sc=False),
        scratch_types=(
            pltpu.VMEM((UPW,), jnp.int32),
            pltpu.VMEM((UPW, D), jnp.float32),
            pltpu.VMEM((NG * L,), jnp.int32),
            pltpu.VMEM((NG * L, DP), jnp.int32),
            pltpu.VMEM((NG * L,), jnp.float32),
            pltpu.VMEM((UPW,), jnp.int32),
            pltpu.SemaphoreType.DMA,
            pltpu.SemaphoreType.DMA,
        ),
    )
    return f(user_id, neg_flat, user_tab, item_tab_packed)


def kernel(user_id, neg_item_ids, user_emb_table, item_emb_table):
    vocab = item_emb_table.shape[0]
    item_packed = lax.bitcast_convert_type(
        item_emb_table.astype(jnp.bfloat16).reshape(vocab, DP, 2), jnp.int32)
    scores_flat, sel = _run(user_id.astype(jnp.int32),
                            neg_item_ids.reshape(-1),
                            user_emb_table, item_packed)
    return scores_flat.reshape(B, N), sel.reshape(B, 1)
